# baseline (speedup 1.0000x reference)
import jax
import jax.numpy as jnp
from jax import lax
from jax.experimental import pallas as pl
from jax.experimental.pallas import tpu as pltpu
from jax.experimental.pallas import tpu_sc as plsc

B = 128
G = 2000
D = 512
CLS_ROW = 60697
NC = 2
NS = 16
NW = NC * NS
ROWS_PER_W = B // NW
CHUNK = 80
N_CHUNKS = G // CHUNK
LANES = 16
OUT_ROW = (G + 1) * D


def _tokenizer_body(expr_hbm, gene_hbm, etab_hbm, out_hbm,
                    idxbuf, gbuf, ebuf, clsbuf, sem):
    cid = lax.axis_index("c")
    sid = lax.axis_index("s")
    wid = sid * NC + cid

    pltpu.sync_copy(gene_hbm.at[pl.ds(CLS_ROW * D, D)], clsbuf)

    def per_row(r, carry):
        b = wid * ROWS_PER_W + r
        obase = b * OUT_ROW
        pltpu.sync_copy(clsbuf, out_hbm.at[pl.ds(obase, D)])

        def per_chunk(k, carry2):
            g0 = k * CHUNK
            pltpu.sync_copy(expr_hbm.at[pl.ds(b * G + g0, CHUNK)], idxbuf)
            pltpu.sync_copy(gene_hbm.at[pl.ds(g0 * D, CHUNK * D)], gbuf)
            pltpu.async_copy(etab_hbm.at[idxbuf], ebuf, sem).wait()

            def per_tok(t, carry3):
                for c in range(D // LANES):
                    sl = pl.ds(t * D + c * LANES, LANES)
                    gbuf[sl] = gbuf[sl] + ebuf[t, pl.ds(c * LANES, LANES)]
                return carry3

            lax.fori_loop(0, CHUNK, per_tok, 0)
            pltpu.sync_copy(gbuf, out_hbm.at[pl.ds(obase + (1 + g0) * D,
                                                   CHUNK * D)])
            return carry2

        lax.fori_loop(0, N_CHUNKS, per_chunk, 0)
        return carry

    lax.fori_loop(0, ROWS_PER_W, per_row, 0)


def kernel(expression, gene_table, expr_table):
    expr_flat = expression.astype(jnp.int32).reshape(B * G)
    gene_flat = gene_table.reshape(-1)
    mesh = plsc.VectorSubcoreMesh(core_axis_name="c", subcore_axis_name="s")
    emb = pl.kernel(
        _tokenizer_body,
        mesh=mesh,
        out_type=jax.ShapeDtypeStruct((B * (G + 1) * D,), jnp.float32),
        scratch_types=[
            pltpu.VMEM((CHUNK,), jnp.int32),
            pltpu.VMEM((CHUNK * D,), jnp.float32),
            pltpu.VMEM((CHUNK, D), jnp.float32),
            pltpu.VMEM((D,), jnp.float32),
            pltpu.SemaphoreType.DMA,
        ],
    )(expr_flat, gene_flat, expr_table)
    mask = jnp.ones((B, G + 1), dtype=jnp.float32)
    return emb.reshape(B, G + 1, D), mask


# speedup vs baseline: 1.3560x; 1.3560x over previous
import jax
import jax.numpy as jnp
from jax import lax
from jax.experimental import pallas as pl
from jax.experimental.pallas import tpu as pltpu
from jax.experimental.pallas import tpu_sc as plsc

B = 128
G = 2000
D = 512
E_ROWS = 52
CLS_ROW = 60697
NC = 2
NS = 16
NW = NC * NS
ROWS_PER_W = B // NW
CHUNK = 40
CPR = G // CHUNK
SP_ROWS = 1440
SP_CHUNKS = SP_ROWS // CHUNK
NCHUNKS = ROWS_PER_W * CPR
LANES = 16
OUT_ROW = (G + 1) * D


def _tokenizer_body(expr_hbm, gene_hbm, etab_hbm, out_hbm,
                    idxbuf0, idxbuf1, gbuf0, gbuf1, ebuf0, ebuf1, clsbuf,
                    gene_sp,
                    sem_idx0, sem_idx1, sem_gene0, sem_gene1,
                    sem_gat0, sem_gat1, sem_out0, sem_out1):
    cid = lax.axis_index("c")
    sid = lax.axis_index("s")
    wid = sid * NC + cid
    idxbuf = (idxbuf0, idxbuf1)
    gbuf = (gbuf0, gbuf1)
    ebuf = (ebuf0, ebuf1)
    sem_idx = (sem_idx0, sem_idx1)
    sem_gene = (sem_gene0, sem_gene1)
    sem_gat = (sem_gat0, sem_gat1)
    sem_out = (sem_out0, sem_out1)

    @pl.when(sid == 0)
    def _stage():
        pltpu.sync_copy(gene_hbm.at[pl.ds(0, SP_ROWS * D)], gene_sp)

    pltpu.sync_copy(gene_hbm.at[pl.ds(CLS_ROW * D, D)], clsbuf)
    for r in range(ROWS_PER_W):
        b = wid * ROWS_PER_W + r
        pltpu.sync_copy(clsbuf, out_hbm.at[pl.ds(b * OUT_ROW, D)])

    plsc.subcore_barrier()

    def _issue(j, sl):
        @pl.when(j >= 2)
        def _():
            pltpu.make_async_copy(
                gbuf[sl], out_hbm.at[pl.ds(0, CHUNK * D)], sem_out[sl]
            ).wait()
        b = wid * ROWS_PER_W + j // CPR
        k = lax.rem(j, CPR)
        pltpu.async_copy(
            expr_hbm.at[pl.ds(b * G + k * CHUNK, CHUNK)], idxbuf[sl],
            sem_idx[sl])
        @pl.when(k < SP_CHUNKS)
        def _gene_from_spmem():
            pltpu.async_copy(
                gene_sp.at[pl.ds(k * (CHUNK * D), CHUNK * D)], gbuf[sl],
                sem_gene[sl])

        @pl.when(k >= SP_CHUNKS)
        def _gene_from_hbm():
            pltpu.async_copy(
                gene_hbm.at[pl.ds(k * (CHUNK * D), CHUNK * D)], gbuf[sl],
                sem_gene[sl])
        pltpu.make_async_copy(
            expr_hbm.at[pl.ds(0, CHUNK)], idxbuf[sl], sem_idx[sl]).wait()
        pltpu.async_copy(etab_hbm.at[idxbuf[sl]], ebuf[sl], sem_gat[sl])

    def _compute(j, sl):
        pltpu.make_async_copy(
            gene_hbm.at[pl.ds(0, CHUNK * D)], gbuf[sl], sem_gene[sl]).wait()
        pltpu.make_async_copy(
            etab_hbm.at[pl.ds(0, CHUNK), :], ebuf[sl], sem_gat[sl]).wait()

        def per_tok(t, carry):
            base = t * D
            for c in range(D // LANES):
                plsc.addupdate(
                    gbuf[sl].at[pl.ds(base + c * LANES, LANES)],
                    ebuf[sl][t, pl.ds(c * LANES, LANES)])
            return carry

        lax.fori_loop(0, CHUNK, per_tok, 0)
        b = wid * ROWS_PER_W + j // CPR
        k = lax.rem(j, CPR)
        pltpu.async_copy(
            gbuf[sl],
            out_hbm.at[pl.ds(b * OUT_ROW + (1 + k * CHUNK) * D, CHUNK * D)],
            sem_out[sl])

    _issue(0, 0)

    def outer(i, carry):
        for par in range(2):
            j = i * 2 + par
            @pl.when(j < NCHUNKS - 1)
            def _():
                _issue(j + 1, (par + 1) % 2)
            _compute(j, par)
        return carry

    lax.fori_loop(0, NCHUNKS // 2, outer, 0)

    for sl in range(2):
        pltpu.make_async_copy(
            gbuf[sl], out_hbm.at[pl.ds(0, CHUNK * D)], sem_out[sl]).wait()


def kernel(expression, gene_table, expr_table):
    expr_flat = expression.astype(jnp.int32).reshape(B * G)
    gene_flat = gene_table.reshape(-1)
    mesh = plsc.VectorSubcoreMesh(core_axis_name="c", subcore_axis_name="s")
    emb = pl.kernel(
        _tokenizer_body,
        mesh=mesh,
        out_type=jax.ShapeDtypeStruct((B * (G + 1) * D,), jnp.float32),
        scratch_types=[
            pltpu.VMEM((CHUNK,), jnp.int32),
            pltpu.VMEM((CHUNK,), jnp.int32),
            pltpu.VMEM((CHUNK * D,), jnp.float32),
            pltpu.VMEM((CHUNK * D,), jnp.float32),
            pltpu.VMEM((CHUNK, D), jnp.float32),
            pltpu.VMEM((CHUNK, D), jnp.float32),
            pltpu.VMEM((D,), jnp.float32),
            pltpu.VMEM_SHARED((SP_ROWS * D,), jnp.float32),
            pltpu.SemaphoreType.DMA,
            pltpu.SemaphoreType.DMA,
            pltpu.SemaphoreType.DMA,
            pltpu.SemaphoreType.DMA,
            pltpu.SemaphoreType.DMA,
            pltpu.SemaphoreType.DMA,
            pltpu.SemaphoreType.DMA,
            pltpu.SemaphoreType.DMA,
        ],
    )(expr_flat, gene_flat, expr_table)
    mask = jnp.ones((B, G + 1), dtype=jnp.float32)
    return emb.reshape(B, G + 1, D), mask


# speedup vs baseline: 1.7389x; 1.2823x over previous
import jax
import jax.numpy as jnp
from jax import lax
from jax.experimental import pallas as pl
from jax.experimental.pallas import tpu as pltpu
from jax.experimental.pallas import tpu_sc as plsc

B = 128
G = 2000
P = G + 1
PPAD = 2016
D = 512
E_ROWS = 53
CLS_ROW = 60697
NC = 2
NS = 16
NW = NC * NS
RPW = B // NW
C = 32
FULL = P // C
TAIL = P - FULL * C
LANES = 16


def _body(eidx_hbm, gtab_hbm, etab_hbm, out_hbm,
          idxall, gbuf0, gbuf1, eb0, eb1, eb2, eb3, tailbuf,
          sem_idx, sem_gene0, sem_gene1,
          sg0, sg1, sg2, sg3, so0, so1, so2, so3):
    cid = lax.axis_index("c")
    sid = lax.axis_index("s")
    wid = sid * NC + cid
    gbuf = (gbuf0, gbuf1)
    ebuf = (eb0, eb1, eb2, eb3)
    sem_gene = (sem_gene0, sem_gene1)
    sem_gat = (sg0, sg1, sg2, sg3)
    sem_out = (so0, so1, so2, so3)

    def issue_gene(k, kk, rows):
        pltpu.async_copy(gtab_hbm.at[pl.ds(k * C, rows), :],
                         gbuf[kk].at[pl.ds(0, rows), :], sem_gene[kk])

    def wait_gene(kk, rows):
        pltpu.make_async_copy(gtab_hbm.at[pl.ds(0, rows), :],
                              gbuf[kk].at[pl.ds(0, rows), :],
                              sem_gene[kk]).wait()

    def issue_gather(k, r, rows):
        idx = idxall.at[pl.ds(r * PPAD + k * C, rows)]
        pltpu.async_copy(etab_hbm.at[idx],
                         ebuf[r].at[pl.ds(0, rows), :], sem_gat[r])

    def wait_gather(r, rows):
        pltpu.make_async_copy(gtab_hbm.at[pl.ds(0, rows), :],
                              ebuf[r].at[pl.ds(0, rows), :],
                              sem_gat[r]).wait()

    def issue_write(k, r, rows):
        b = wid * RPW + r
        pltpu.async_copy(ebuf[r].at[pl.ds(0, rows), :],
                         out_hbm.at[b, pl.ds(k * C, rows), :], sem_out[r])

    def wait_write(r, rows):
        pltpu.make_async_copy(ebuf[r].at[pl.ds(0, rows), :],
                              out_hbm.at[0, pl.ds(0, rows), :],
                              sem_out[r]).wait()

    def add_rows(kk, r, rows):
        def per_tok(t, carry):
            for c in range(D // LANES):
                sl = pl.ds(c * LANES, LANES)
                plsc.addupdate(ebuf[r].at[t, sl], gbuf[kk][t, sl])
            return carry
        lax.fori_loop(0, rows, per_tok, 0)

    pltpu.sync_copy(eidx_hbm.at[pl.ds(wid * RPW * PPAD, RPW * PPAD)], idxall)
    issue_gene(0, 0, C)
    issue_gather(0, 0, C)
    issue_gather(0, 1, C)

    issue_gene(1, 1, C)
    wait_gene(0, C)
    for r in range(4):
        if r >= 2:
            wait_write((r + 2) % 4, C)
        nk, nr = (0, r + 2) if r < 2 else (1, (r + 2) % 4)
        issue_gather(nk, nr, C)
        wait_gather(r, C)
        add_rows(0, r, C)
        issue_write(0, r, C)

    def steady(jo, carry):
        for kk_off in range(2):
            k = 1 + jo * 2 + kk_off
            kk = (1 + kk_off) % 2
            issue_gene(k + 1, (kk + 1) % 2, C)
            wait_gene(kk, C)
            for r in range(4):
                wait_write((r + 2) % 4, C)
                nk = k if r < 2 else k + 1
                issue_gather(nk, (r + 2) % 4, C)
                wait_gather(r, C)
                add_rows(kk, r, C)
                issue_write(k, r, C)
        return carry

    lax.fori_loop(0, 30, steady, 0)

    issue_gene(62, 0, C)
    wait_gene(1, C)
    for r in range(4):
        wait_write((r + 2) % 4, C)
        issue_gather(61 if r < 2 else 62, (r + 2) % 4, C)
        wait_gather(r, C)
        add_rows(1, r, C)
        issue_write(61, r, C)

    wait_gene(0, C)
    for r in range(4):
        if r < 2:
            wait_write(r + 2, C)
            issue_gather(62, r + 2, C)
        wait_gather(r, C)

        def tail_tok(t, carry):
            for c in range(D // LANES):
                sl = pl.ds(c * LANES, LANES)
                tailbuf[t, sl] = gbuf0[t, sl] + ebuf[r][t, sl]
            return carry

        lax.fori_loop(0, TAIL, tail_tok, 0)
        b = wid * RPW + r
        pltpu.sync_copy(tailbuf, out_hbm.at[b, pl.ds(FULL * C, TAIL), :])


def kernel(expression, gene_table, expr_table):
    expr_i = expression.astype(jnp.int32)
    gtab = jnp.concatenate(
        [gene_table[CLS_ROW:CLS_ROW + 1], gene_table[:G],
         jnp.zeros((PPAD - P, D), jnp.float32)], axis=0)
    etab = jnp.concatenate(
        [expr_table, jnp.zeros((1, D), jnp.float32)], axis=0)
    zero_col = jnp.full((B, 1), E_ROWS - 1, jnp.int32)
    pad_cols = jnp.full((B, PPAD - P), E_ROWS - 1, jnp.int32)
    eidx = jnp.concatenate([zero_col, expr_i, pad_cols], axis=1)
    eidx_flat = eidx.reshape(B * PPAD)

    mesh = plsc.VectorSubcoreMesh(core_axis_name="c", subcore_axis_name="s")
    emb = pl.kernel(
        _body,
        mesh=mesh,
        out_type=jax.ShapeDtypeStruct((B, P, D), jnp.float32),
        scratch_types=[
            pltpu.VMEM((RPW * PPAD,), jnp.int32),
            pltpu.VMEM((C, D), jnp.float32),
            pltpu.VMEM((C, D), jnp.float32),
            pltpu.VMEM((C, D), jnp.float32),
            pltpu.VMEM((C, D), jnp.float32),
            pltpu.VMEM((C, D), jnp.float32),
            pltpu.VMEM((C, D), jnp.float32),
            pltpu.VMEM((TAIL, D), jnp.float32),
            pltpu.SemaphoreType.DMA,
            pltpu.SemaphoreType.DMA,
            pltpu.SemaphoreType.DMA,
            pltpu.SemaphoreType.DMA,
            pltpu.SemaphoreType.DMA,
            pltpu.SemaphoreType.DMA,
            pltpu.SemaphoreType.DMA,
            pltpu.SemaphoreType.DMA,
            pltpu.SemaphoreType.DMA,
            pltpu.SemaphoreType.DMA,
            pltpu.SemaphoreType.DMA,
        ],
    )(eidx_flat, gtab, etab)
    mask = jnp.ones((B, P), dtype=jnp.float32)
    return emb, mask


# speedup vs baseline: 1.7455x; 1.0038x over previous
import jax
import jax.numpy as jnp
from jax import lax
from jax.experimental import pallas as pl
from jax.experimental.pallas import tpu as pltpu
from jax.experimental.pallas import tpu_sc as plsc

B = 128
G = 2000
P = G + 1
PPAD = 2016
D = 512
E_ROWS = 53
CLS_ROW = 60697
NC = 2
NS = 16
NW = NC * NS
RPW = B // NW
C = 32
NCH = PPAD // C
TAIL = P - (NCH - 1) * C
LANES = 16


def _body(eidx_hbm, gtab_hbm, etab_hbm, out_hbm,
          idxall, etab_v, gbuf0, gbuf1, ob0, ob1, tailbuf, erow_s,
          sem_gene0, sem_gene1, sem_out0, sem_out1):
    cid = lax.axis_index("c")
    sid = lax.axis_index("s")
    wid = sid * NC + cid
    gbuf = (gbuf0, gbuf1)
    obuf = (ob0, ob1)
    sem_gene = (sem_gene0, sem_gene1)
    sem_out = (sem_out0, sem_out1)
    iota = lax.iota(jnp.int32, LANES)

    def issue_gene(k, kk):
        pltpu.async_copy(gtab_hbm.at[pl.ds(k * C, C), :], gbuf[kk],
                         sem_gene[kk])

    def wait_gene(kk):
        pltpu.make_async_copy(gtab_hbm.at[pl.ds(0, C), :], gbuf[kk],
                              sem_gene[kk]).wait()

    def issue_write(k, r, s):
        b = wid * RPW + r
        pltpu.async_copy(obuf[s], out_hbm.at[b, pl.ds(k * C, C), :],
                         sem_out[s])

    def wait_write(s):
        pltpu.make_async_copy(obuf[s], out_hbm.at[0, pl.ds(0, C), :],
                              sem_out[s]).wait()

    def compute(k, r, kk, s):
        idx_base = r * PPAD + k * C
        for g in range(C // LANES):
            ev = idxall[pl.ds(idx_base + g * LANES, LANES)]
            for tt in range(LANES):
                erow_s[g * LANES + tt] = ev[tt]

        def per_tok(t, carry):
            e_row = erow_s[t]
            for c in range(D // LANES):
                sl = pl.ds(c * LANES, LANES)
                obuf[s][t, sl] = gbuf[kk][t, sl] + etab_v[e_row, sl]
            return carry

        lax.fori_loop(0, C, per_tok, 0)

    pltpu.sync_copy(eidx_hbm.at[pl.ds(wid * RPW * PPAD, RPW * PPAD)], idxall)
    pltpu.sync_copy(etab_hbm, etab_v)
    issue_gene(0, 0)

    issue_gene(1, 1)
    wait_gene(0)
    for r in range(4):
        if r >= 2:
            wait_write(r % 2)
        compute(0, r, 0, r % 2)
        issue_write(0, r, r % 2)

    def steady(jo, carry):
        for kk_off in range(2):
            k = 1 + jo * 2 + kk_off
            kk = (1 + kk_off) % 2
            issue_gene(k + 1, (kk + 1) % 2)
            wait_gene(kk)
            for r in range(4):
                wait_write(r % 2)
                compute(k, r, kk, r % 2)
                issue_write(k, r, r % 2)
        return carry

    lax.fori_loop(0, 30, steady, 0)

    issue_gene(62, 0)
    wait_gene(1)
    for r in range(4):
        wait_write(r % 2)
        compute(61, r, 1, r % 2)
        issue_write(61, r, r % 2)

    wait_gene(0)
    for r in range(4):
        if r < 2:
            wait_write(r % 2)
        compute(62, r, 0, r % 2)

        def tail_tok(t, carry):
            for c in range(D // LANES):
                sl = pl.ds(c * LANES, LANES)
                tailbuf[t, sl] = obuf[r % 2][t, sl]
            return carry

        lax.fori_loop(0, TAIL, tail_tok, 0)
        b = wid * RPW + r
        pltpu.sync_copy(tailbuf, out_hbm.at[b, pl.ds((NCH - 1) * C, TAIL), :])


def kernel(expression, gene_table, expr_table):
    expr_i = expression.astype(jnp.int32)
    gtab = jnp.concatenate(
        [gene_table[CLS_ROW:CLS_ROW + 1], gene_table[:G],
         jnp.zeros((PPAD - P, D), jnp.float32)], axis=0)
    etab2 = jnp.concatenate(
        [expr_table, jnp.zeros((1, D), jnp.float32)], axis=0)
    zero_col = jnp.full((B, 1), E_ROWS - 1, jnp.int32)
    pad_cols = jnp.full((B, PPAD - P), E_ROWS - 1, jnp.int32)
    eidx = jnp.concatenate([zero_col, expr_i, pad_cols], axis=1)
    eidx_flat = eidx.reshape(B * PPAD)

    mesh = plsc.VectorSubcoreMesh(core_axis_name="c", subcore_axis_name="s")
    emb = pl.kernel(
        _body,
        mesh=mesh,
        out_type=jax.ShapeDtypeStruct((B, P, D), jnp.float32),
        scratch_types=[
            pltpu.VMEM((RPW * PPAD,), jnp.int32),
            pltpu.VMEM((E_ROWS, D), jnp.float32),
            pltpu.VMEM((C, D), jnp.float32),
            pltpu.VMEM((C, D), jnp.float32),
            pltpu.VMEM((C, D), jnp.float32),
            pltpu.VMEM((C, D), jnp.float32),
            pltpu.VMEM((TAIL, D), jnp.float32),
            pltpu.SMEM((C,), jnp.int32),
            pltpu.SemaphoreType.DMA,
            pltpu.SemaphoreType.DMA,
            pltpu.SemaphoreType.DMA,
            pltpu.SemaphoreType.DMA,
        ],
    )(eidx_flat, gtab, etab2)
    mask = jnp.ones((B, P), dtype=jnp.float32)
    return emb, mask


# speedup vs baseline: 2.0499x; 1.1744x over previous
import jax
import jax.numpy as jnp
from jax import lax
from jax.experimental import pallas as pl
from jax.experimental.pallas import tpu as pltpu
from jax.experimental.pallas import tpu_sc as plsc

B = 128
G = 2000
P = G + 1
PPAD = 2016
D = 512
E_ROWS = 53
CLS_ROW = 60697
NC = 2
NS = 16
NW = NC * NS
RPW = B // NW
C = 32
NCH = PPAD // C
TAIL = P - (NCH - 1) * C
LANES = 16


def _body(eidx_hbm, gtab_hbm, etab_hbm, out_hbm,
          idxall, etab_v, eb0, eb1, eb2, eb3, tailbuf, erow_s,
          sg0, sg1, sg2, sg3, so0, so1, so2, so3):
    cid = lax.axis_index("c")
    sid = lax.axis_index("s")
    wid = sid * NC + cid
    ebuf = (eb0, eb1, eb2, eb3)
    sem_gene = (sg0, sg1, sg2, sg3)
    sem_out = (so0, so1, so2, so3)

    def issue_gene(k, q):
        pltpu.async_copy(gtab_hbm.at[pl.ds(k * C, C), :], ebuf[q],
                         sem_gene[q])

    def wait_gene(q):
        pltpu.make_async_copy(gtab_hbm.at[pl.ds(0, C), :], ebuf[q],
                              sem_gene[q]).wait()

    def issue_write(k, r, q):
        b = wid * RPW + r
        pltpu.async_copy(ebuf[q], out_hbm.at[b, pl.ds(k * C, C), :],
                         sem_out[q])

    def wait_write(q):
        pltpu.make_async_copy(ebuf[q], out_hbm.at[0, pl.ds(0, C), :],
                              sem_out[q]).wait()

    def accumulate(k, r, q):
        idx_base = r * PPAD + k * C
        for g in range(C // LANES):
            ev = idxall[pl.ds(idx_base + g * LANES, LANES)]
            for tt in range(LANES):
                erow_s[g * LANES + tt] = ev[tt]

        def per_tok(t, carry):
            e_row = erow_s[t]
            for c in range(D // LANES):
                sl = pl.ds(c * LANES, LANES)
                plsc.addupdate(ebuf[q].at[t, sl], etab_v[e_row, sl])
            return carry

        lax.fori_loop(0, C, per_tok, 0)

    pltpu.sync_copy(eidx_hbm.at[pl.ds(wid * RPW * PPAD, RPW * PPAD)], idxall)
    pltpu.sync_copy(etab_hbm, etab_v)
    issue_gene(0, 0)
    issue_gene(0, 1)

    for r in range(4):
        if r >= 2:
            wait_write((r + 2) % 4)
        issue_gene(0 if r < 2 else 1, (r + 2) % 4)
        wait_gene(r)
        accumulate(0, r, r)
        issue_write(0, r, r)

    def steady(k, carry):
        for r in range(4):
            wait_write((r + 2) % 4)
            issue_gene(k if r < 2 else k + 1, (r + 2) % 4)
            wait_gene(r)
            accumulate(k, r, r)
            issue_write(k, r, r)
        return carry

    lax.fori_loop(1, NCH - 1, steady, 0)

    for r in range(4):
        if r < 2:
            wait_write(r + 2)
            issue_gene(NCH - 1, r + 2)
        wait_gene(r)
        accumulate(NCH - 1, r, r)

        def tail_tok(t, carry):
            for c in range(D // LANES):
                sl = pl.ds(c * LANES, LANES)
                tailbuf[t, sl] = ebuf[r][t, sl]
            return carry

        lax.fori_loop(0, TAIL, tail_tok, 0)
        b = wid * RPW + r
        pltpu.sync_copy(tailbuf, out_hbm.at[b, pl.ds((NCH - 1) * C, TAIL), :])


def kernel(expression, gene_table, expr_table):
    expr_i = expression.astype(jnp.int32)
    gtab = jnp.concatenate(
        [gene_table[CLS_ROW:CLS_ROW + 1], gene_table[:G],
         jnp.zeros((PPAD - P, D), jnp.float32)], axis=0)
    etab2 = jnp.concatenate(
        [expr_table, jnp.zeros((1, D), jnp.float32)], axis=0)
    zero_col = jnp.full((B, 1), E_ROWS - 1, jnp.int32)
    pad_cols = jnp.full((B, PPAD - P), E_ROWS - 1, jnp.int32)
    eidx = jnp.concatenate([zero_col, expr_i, pad_cols], axis=1)
    eidx_flat = eidx.reshape(B * PPAD)

    mesh = plsc.VectorSubcoreMesh(core_axis_name="c", subcore_axis_name="s")
    emb = pl.kernel(
        _body,
        mesh=mesh,
        out_type=jax.ShapeDtypeStruct((B, P, D), jnp.float32),
        scratch_types=[
            pltpu.VMEM((RPW * PPAD,), jnp.int32),
            pltpu.VMEM((E_ROWS, D), jnp.float32),
            pltpu.VMEM((C, D), jnp.float32),
            pltpu.VMEM((C, D), jnp.float32),
            pltpu.VMEM((C, D), jnp.float32),
            pltpu.VMEM((C, D), jnp.float32),
            pltpu.VMEM((TAIL, D), jnp.float32),
            pltpu.SMEM((C,), jnp.int32),
            pltpu.SemaphoreType.DMA,
            pltpu.SemaphoreType.DMA,
            pltpu.SemaphoreType.DMA,
            pltpu.SemaphoreType.DMA,
            pltpu.SemaphoreType.DMA,
            pltpu.SemaphoreType.DMA,
            pltpu.SemaphoreType.DMA,
            pltpu.SemaphoreType.DMA,
        ],
    )(eidx_flat, gtab, etab2)
    mask = jnp.ones((B, P), dtype=jnp.float32)
    return emb, mask
